# Initial kernel scaffold; baseline (speedup 1.0000x reference)
import jax
import jax.numpy as jnp
from jax.experimental import pallas as pl


def kernel(x, W_in, b_in, W_m1, b_m1, W_m2, b_m2, ln_g, ln_b, W_r1, b_r1, W_r2, b_r2):
    raise NotImplementedError("write your pallas kernel here")


# speedup vs baseline: 11.4476x; 11.4476x over previous
import functools

import jax
import jax.numpy as jnp
from jax.experimental import pallas as pl
from jax.experimental.pallas import tpu as pltpu

K = 32
TEMP = 0.1
STEPS = 3
LN_EPS = 1e-5


def _silu(v):
    return v * (1.0 / (1.0 + jnp.exp(-v)))


def _knn_body(x_ref, w_in_ref, b_in_ref, m_ref, h0_ref, *, rb, n, k):
    xb = x_ref[pl.ds(pl.program_id(0) * rb, rb), :]
    xall = x_ref[...]
    nb = jnp.sum(xb * xb, axis=1, keepdims=True)
    na = jnp.sum(xall * xall, axis=1)[None, :]
    cross = jax.lax.dot_general(
        xb, xall, (((1,), (1,)), ((), ())), preferred_element_type=jnp.float32
    )
    sq = jnp.maximum(nb + na - 2.0 * cross, 0.0)
    dist = jnp.sqrt(jnp.maximum(sq, 1e-12))
    rows = pl.program_id(0) * rb + jax.lax.broadcasted_iota(jnp.int32, (rb, n), 0)
    cols = jax.lax.broadcasted_iota(jnp.int32, (rb, n), 1)
    dist = jnp.where(rows == cols, jnp.inf, dist)

    bits = jax.lax.bitcast_convert_type(dist, jnp.int32)

    def search_body(_, lohi):
        lo, hi = lohi
        mid = lo + jax.lax.div(hi - lo, 2)
        cnt = jnp.sum((bits <= mid[:, None]).astype(jnp.float32), axis=1)
        ge = cnt >= float(k)
        return jnp.where(ge, lo, mid + 1), jnp.where(ge, mid, hi)

    lo0 = jnp.zeros((rb,), jnp.int32)
    hi0 = jnp.full((rb,), jnp.int32(0x7F800000))
    _, hi = jax.lax.fori_loop(0, 31, search_body, (lo0, hi0))
    t = jax.lax.bitcast_convert_type(hi, jnp.float32)

    m = jnp.min(dist, axis=1, keepdims=True)
    mask = dist <= t[:, None]
    e = jnp.where(mask, jnp.exp(-(dist - m) / TEMP), 0.0)
    den = jnp.sum(e, axis=1, keepdims=True)
    m_ref[...] = e / den

    h0_ref[...] = (
        jax.lax.dot_general(
            xb, w_in_ref[...], (((1,), (0,)), ((), ())),
            preferred_element_type=jnp.float32,
        )
        + b_in_ref[...]
    )


def _step_body(m_ref, hid_ref, hb_ref, xb_ref, w1h_ref, w1a_ref, w1x_ref,
               b1_ref, w2_ref, b2_ref, g_ref, b_ref, out_ref):
    def mm(a, b):
        return jax.lax.dot_general(
            a, b, (((1,), (0,)), ((), ())), preferred_element_type=jnp.float32
        )

    agg = mm(m_ref[...], hid_ref[...])
    hb = hb_ref[...]
    z = mm(hb, w1h_ref[...]) + mm(agg, w1a_ref[...]) + mm(xb_ref[...], w1x_ref[...])
    z = z + b1_ref[...]
    msg = mm(_silu(z), w2_ref[...]) + b2_ref[...]
    pre = hb + msg
    mu = jnp.mean(pre, axis=1, keepdims=True)
    var = jnp.mean((pre - mu) ** 2, axis=1, keepdims=True)
    out_ref[...] = (pre - mu) / jnp.sqrt(var + LN_EPS) * g_ref[...] + b_ref[...]


def _readout_body(hb_ref, xb_ref, w1h_ref, w1x_ref, b1_ref, w2_ref, b2_ref,
                  out_ref):
    def mm(a, b):
        return jax.lax.dot_general(
            a, b, (((1,), (0,)), ((), ())), preferred_element_type=jnp.float32
        )

    z = mm(hb_ref[...], w1h_ref[...]) + mm(xb_ref[...], w1x_ref[...]) + b1_ref[...]
    h1 = _silu(z)
    ro = jnp.sum(h1 * w2_ref[...].T, axis=1) + b2_ref[0]
    out_ref[...] = jnp.maximum(ro, 0.0) + jnp.log1p(jnp.exp(-jnp.abs(ro)))


def kernel(x, W_in, b_in, W_m1, b_m1, W_m2, b_m2, ln_g, ln_b, W_r1, b_r1,
           W_r2, b_r2):
    n, f = x.shape
    h = W_in.shape[1]
    rb = min(256, n)
    grid = (n // rb,)

    full = lambda shape: pl.BlockSpec(shape, lambda i: (0,) * len(shape))
    rowblk = lambda shape: pl.BlockSpec(shape, lambda i: (i,) + (0,) * (len(shape) - 1))

    M, hidden = pl.pallas_call(
        functools.partial(_knn_body, rb=rb, n=n, k=K),
        grid=grid,
        in_specs=[full((n, f)), full((f, h)), full((h,))],
        out_specs=[rowblk((rb, n)), rowblk((rb, h))],
        out_shape=[
            jax.ShapeDtypeStruct((n, n), jnp.float32),
            jax.ShapeDtypeStruct((n, h), jnp.float32),
        ],
    )(x, W_in, b_in)

    W1h, W1a, W1x = W_m1[:h], W_m1[h:2 * h], W_m1[2 * h:]
    step = pl.pallas_call(
        _step_body,
        grid=grid,
        in_specs=[rowblk((rb, n)), full((n, h)), rowblk((rb, h)), rowblk((rb, f)),
                  full((h, h)), full((h, h)), full((f, h)), full((h,)),
                  full((h, h)), full((h,)), full((h,)), full((h,))],
        out_specs=rowblk((rb, h)),
        out_shape=jax.ShapeDtypeStruct((n, h), jnp.float32),
    )
    for _ in range(STEPS):
        hidden = step(M, hidden, hidden, x, W1h, W1a, W1x, b_m1, W_m2, b_m2,
                      ln_g, ln_b)

    Wr1h, Wr1x = W_r1[:h], W_r1[h:]
    out = pl.pallas_call(
        _readout_body,
        grid=grid,
        in_specs=[rowblk((rb, h)), rowblk((rb, f)), full((h, h)), full((f, h)),
                  full((h,)), full((h, 1)), full((1,))],
        out_specs=rowblk((rb,)),
        out_shape=jax.ShapeDtypeStruct((n,), jnp.float32),
    )(hidden, x, Wr1h, Wr1x, b_r1, W_r2, b_r2)
    return out
